# baseline (speedup 1.0000x reference)
import functools

import jax
import jax.numpy as jnp
from jax.experimental import pallas as pl
from jax.experimental.pallas import tpu as pltpu

_SLOPE = 0.01
_SELF_LOGIT = 10.0
_NEG = -1e30


def _ceil_to(n, m):
    return ((n + m - 1) // m) * m


def _pair_mlp_kernel(sub_ref, obj_ref, feats_ref, conf_row_ref, conf_col_ref,
                     w1t_ref, b1_ref, scale_ref, shift_ref, w2t_ref, b2_ref,
                     ra_ref, rb_ref, x2_scr, *, tp, d, dh):
    i = pl.program_id(0)
    base = i * tp
    iota8 = jax.lax.broadcasted_iota(jnp.int32, (8, d), 0)
    m8 = [iota8 == k for k in range(8)]
    for pc in range(tp // 8):
        ga = gb = None
        for k in range(8):
            g = base + pc * 8 + k
            s = sub_ref[g]
            o = obj_ref[g]
            s8 = pl.multiple_of((s >> 3) << 3, 8)
            o8 = pl.multiple_of((o >> 3) << 3, 8)
            a_r = pltpu.roll(feats_ref[pl.ds(s8, 8), :], (k - s) & 7, axis=0)
            b_r = pltpu.roll(feats_ref[pl.ds(o8, 8), :], (k - o) & 7, axis=0)
            ga = a_r if k == 0 else jnp.where(m8[k], a_r, ga)
            gb = b_r if k == 0 else jnp.where(m8[k], b_r, gb)
        x2_scr[pl.ds(pc * 8, 8), 0:d] = ga
        x2_scr[pl.ds(pc * 8, 8), d:2 * d] = gb

    hv = (jnp.dot(x2_scr[...], w1t_ref[...], preferred_element_type=jnp.float32)
          + b1_ref[...])
    hv = jnp.where(hv >= 0, hv, _SLOPE * hv)
    hv = hv * scale_ref[...] + shift_ref[...]
    rel = jnp.dot(hv, w2t_ref[...], preferred_element_type=jnp.float32) + b2_ref[...]

    cmax = jnp.maximum(jnp.max(conf_row_ref[...]), _SELF_LOGIT)
    e_col = jnp.exp(conf_col_ref[...] - cmax)
    ra_ref[:, :dh] = rel[:, :dh] * e_col
    rb_ref[:, :dh] = rel[:, dh:] * e_col
    e_b = jnp.broadcast_to(e_col, (tp, 128))
    ra_ref[:, dh:] = e_b
    rb_ref[:, dh:] = e_b


def _pair_mlp(sub, obj, feats_p, conf_row, conf_col,
              w1t, b1, scale, shift, w2t, b2, *, tp):
    p_pad = conf_col.shape[0]
    o_pad, d = feats_p.shape
    hid = w1t.shape[1]
    dout = w2t.shape[1]
    dh = dout // 2
    w = dh + 128
    return pl.pallas_call(
        functools.partial(_pair_mlp_kernel, tp=tp, d=d, dh=dh),
        out_shape=(jax.ShapeDtypeStruct((p_pad, w), jnp.float32),
                   jax.ShapeDtypeStruct((p_pad, w), jnp.float32)),
        grid_spec=pltpu.PrefetchScalarGridSpec(
            num_scalar_prefetch=2,
            grid=(p_pad // tp,),
            in_specs=[
                pl.BlockSpec((o_pad, d), lambda i, *_: (0, 0)),
                pl.BlockSpec((1, p_pad), lambda i, *_: (0, 0)),
                pl.BlockSpec((tp, 1), lambda i, *_: (i, 0)),
                pl.BlockSpec((2 * d, hid), lambda i, *_: (0, 0)),
                pl.BlockSpec((1, hid), lambda i, *_: (0, 0)),
                pl.BlockSpec((1, hid), lambda i, *_: (0, 0)),
                pl.BlockSpec((1, hid), lambda i, *_: (0, 0)),
                pl.BlockSpec((hid, dout), lambda i, *_: (0, 0)),
                pl.BlockSpec((1, dout), lambda i, *_: (0, 0)),
            ],
            out_specs=[pl.BlockSpec((tp, w), lambda i, *_: (i, 0)),
                       pl.BlockSpec((tp, w), lambda i, *_: (i, 0))],
            scratch_shapes=[pltpu.VMEM((tp, 2 * d), jnp.float32)],
        ),
        compiler_params=pltpu.CompilerParams(
            dimension_semantics=("parallel",),
            vmem_limit_bytes=44 * 1024 * 1024),
    )(sub, obj, feats_p, conf_row, conf_col, w1t, b1, scale, shift, w2t, b2)


def _scatter_combine_kernel(sub_ref, obj_ref, ra_ref, rb_ref, x_ref,
                            conf_row_ref, out_ref, acc_a, acc_b,
                            *, tp, to, np_, dh):
    i = pl.program_id(0)
    w = acc_a.shape[2]

    @pl.when(i == 0)
    def _init():
        acc_a[...] = jnp.zeros(acc_a.shape, jnp.float32)
        acc_b[...] = jnp.zeros(acc_b.shape, jnp.float32)

    @pl.when(i < np_)
    def _scatter_phase():
        base = i * tp
        im = jax.lax.broadcasted_iota(jnp.int32, (8, w), 0)
        for pc in range(tp // 8):
            ca = ra_ref[pc * 8:(pc + 1) * 8, :]
            cb = rb_ref[pc * 8:(pc + 1) * 8, :]
            for k in range(8):
                g = base + pc * 8 + k
                s = sub_ref[g]
                o = obj_ref[g]
                sc = s >> 3
                sl = s & 7
                oc = o >> 3
                ol = o & 7
                add_a = jnp.where(im == sl, pltpu.roll(ca, sl - k, axis=0), 0.0)
                add_b = jnp.where(im == ol, pltpu.roll(cb, ol - k, axis=0), 0.0)
                acc_a[pl.ds(sc, 1)] = acc_a[pl.ds(sc, 1)] + add_a[None]
                acc_b[pl.ds(oc, 1)] = acc_b[pl.ds(oc, 1)] + add_b[None]

    @pl.when(i >= np_)
    def _combine_phase():
        t = i - np_
        cmax = jnp.maximum(jnp.max(conf_row_ref[...]), _SELF_LOGIT)
        w_self = jnp.exp(_SELF_LOGIT - cmax)
        c8 = to // 8
        a_sl = acc_a[pl.ds(t * c8, c8)].reshape(to, w)
        b_sl = acc_b[pl.ds(t * c8, c8)].reshape(to, w)
        num = a_sl[:, :dh] + b_sl[:, :dh]
        den = a_sl[:, dh:dh + 1] + b_sl[:, dh:dh + 1]
        out_ref[...] = (w_self * x_ref[...] + num) / (w_self + den)


def _scatter_combine(sub, obj, ra, rb, feats_p, conf_row, *, tp, to):
    p_pad, w = ra.shape
    o_pad, dh = feats_p.shape
    np_ = p_pad // tp
    no_ = o_pad // to
    return pl.pallas_call(
        functools.partial(_scatter_combine_kernel, tp=tp, to=to, np_=np_,
                          dh=dh),
        out_shape=jax.ShapeDtypeStruct((o_pad, dh), jnp.float32),
        grid_spec=pltpu.PrefetchScalarGridSpec(
            num_scalar_prefetch=2,
            grid=(np_ + no_,),
            in_specs=[
                pl.BlockSpec((tp, w),
                             lambda i, *_: (jnp.minimum(i, np_ - 1), 0)),
                pl.BlockSpec((tp, w),
                             lambda i, *_: (jnp.minimum(i, np_ - 1), 0)),
                pl.BlockSpec((to, dh),
                             lambda i, *_: (jnp.maximum(i - np_, 0), 0)),
                pl.BlockSpec((1, p_pad), lambda i, *_: (0, 0)),
            ],
            out_specs=pl.BlockSpec((to, dh),
                                   lambda i, *_: (jnp.maximum(i - np_, 0), 0)),
            scratch_shapes=[pltpu.VMEM((o_pad // 8, 8, w), jnp.float32),
                            pltpu.VMEM((o_pad // 8, 8, w), jnp.float32)],
        ),
        compiler_params=pltpu.CompilerParams(
            dimension_semantics=("arbitrary",),
            vmem_limit_bytes=52 * 1024 * 1024),
    )(sub, obj, ra, rb, feats_p, conf_row)


def kernel(w1a, w1b, b1, scale, shift, w2t, b2, object_feats, pairs, confidence):
    o, d = object_feats.shape
    p = pairs.shape[0]
    dout = w2t.shape[1]
    dh = dout // 2

    tp = 512 if p >= 512 else _ceil_to(p, 8)
    to = 1024 if o >= 1024 else _ceil_to(o, 8)
    p_pad = _ceil_to(p, tp)
    o_pad = _ceil_to(o, to)

    feats = object_feats.astype(jnp.float32)
    if o_pad != o:
        feats = jnp.concatenate(
            [feats, jnp.zeros((o_pad - o, d), jnp.float32)], axis=0)
    pr = pairs.astype(jnp.int32)
    conf = confidence.astype(jnp.float32)
    if p_pad != p:
        pr = jnp.concatenate(
            [pr, jnp.zeros((p_pad - p, 2), jnp.int32)], axis=0)
        conf = jnp.concatenate(
            [conf, jnp.full((p_pad - p,), _NEG, jnp.float32)], axis=0)
    sub = pr[:, 0]
    obj = pr[:, 1]
    conf_row = conf[None, :]
    conf_col = conf[:, None]
    w1t = jnp.concatenate([w1a, w1b], axis=0)

    ra, rb = _pair_mlp(sub, obj, feats, conf_row, conf_col,
                       w1t, b1, scale, shift, w2t, b2, tp=tp)
    new_p = _scatter_combine(sub, obj, ra, rb, feats, conf_row, tp=tp, to=to)
    new = new_p[:o].astype(object_feats.dtype)
    return new, pairs, confidence


# speedup vs baseline: 1.0859x; 1.0859x over previous
import functools

import jax
import jax.numpy as jnp
from jax.experimental import pallas as pl
from jax.experimental.pallas import tpu as pltpu

_SLOPE = 0.01
_SELF_LOGIT = 10.0
_NEG = -1e30


def _ceil_to(n, m):
    return ((n + m - 1) // m) * m


def _pair_mlp_kernel(sub_ref, obj_ref, feats_ref, sub_col_ref, obj_col_ref,
                     conf_row_ref, conf_col_ref,
                     w1t_ref, b1_ref, scale_ref, shift_ref, w2t_ref, b2_ref,
                     ra_ref, rb_ref, x2_scr, *, tp, d, dh, gsz):
    i = pl.program_id(0)
    base = i * tp
    iota_g = jax.lax.broadcasted_iota(jnp.int32, (gsz, 1), 0)
    iota_l = jax.lax.broadcasted_iota(jnp.int32, (gsz, 8 * gsz), 1)
    for pc in range(tp // gsz):
        s_col = sub_col_ref[pl.ds(pc * gsz, gsz), :]
        o_col = obj_col_ref[pl.ds(pc * gsz, gsz), :]
        ea = jnp.where(iota_l == ((s_col & 7) + 8 * iota_g), 1.0, 0.0)
        eb = jnp.where(iota_l == ((o_col & 7) + 8 * iota_g), 1.0, 0.0)
        cha = []
        chb = []
        for k in range(gsz):
            g = base + pc * gsz + k
            s = sub_ref[g]
            o = obj_ref[g]
            s8 = pl.multiple_of((s >> 3) << 3, 8)
            o8 = pl.multiple_of((o >> 3) << 3, 8)
            cha.append(feats_ref[pl.ds(s8, 8), :])
            chb.append(feats_ref[pl.ds(o8, 8), :])
        sta = jnp.concatenate(cha, axis=0)
        stb = jnp.concatenate(chb, axis=0)
        x2_scr[pl.ds(pc * gsz, gsz), 0:d] = jnp.dot(
            ea, sta, preferred_element_type=jnp.float32)
        x2_scr[pl.ds(pc * gsz, gsz), d:2 * d] = jnp.dot(
            eb, stb, preferred_element_type=jnp.float32)

    hv = (jnp.dot(x2_scr[...], w1t_ref[...], preferred_element_type=jnp.float32)
          + b1_ref[...])
    hv = jnp.where(hv >= 0, hv, _SLOPE * hv)
    hv = hv * scale_ref[...] + shift_ref[...]
    rel = jnp.dot(hv, w2t_ref[...], preferred_element_type=jnp.float32) + b2_ref[...]

    cmax = jnp.maximum(jnp.max(conf_row_ref[...]), _SELF_LOGIT)
    e_col = jnp.exp(conf_col_ref[...] - cmax)
    ra_ref[:, :dh] = rel[:, :dh] * e_col
    rb_ref[:, :dh] = rel[:, dh:] * e_col
    e_b = jnp.broadcast_to(e_col, (tp, 128))
    ra_ref[:, dh:] = e_b
    rb_ref[:, dh:] = e_b


def _pair_mlp(sub, obj, feats_p, conf_row, conf_col,
              w1t, b1, scale, shift, w2t, b2, *, tp):
    p_pad = conf_col.shape[0]
    o_pad, d = feats_p.shape
    hid = w1t.shape[1]
    dout = w2t.shape[1]
    dh = dout // 2
    w = dh + 128
    gsz = 16 if tp % 16 == 0 else 8
    return pl.pallas_call(
        functools.partial(_pair_mlp_kernel, tp=tp, d=d, dh=dh, gsz=gsz),
        out_shape=(jax.ShapeDtypeStruct((p_pad, w), jnp.float32),
                   jax.ShapeDtypeStruct((p_pad, w), jnp.float32)),
        grid_spec=pltpu.PrefetchScalarGridSpec(
            num_scalar_prefetch=2,
            grid=(p_pad // tp,),
            in_specs=[
                pl.BlockSpec((o_pad, d), lambda i, *_: (0, 0)),
                pl.BlockSpec((tp, 1), lambda i, *_: (i, 0)),
                pl.BlockSpec((tp, 1), lambda i, *_: (i, 0)),
                pl.BlockSpec((1, p_pad), lambda i, *_: (0, 0)),
                pl.BlockSpec((tp, 1), lambda i, *_: (i, 0)),
                pl.BlockSpec((2 * d, hid), lambda i, *_: (0, 0)),
                pl.BlockSpec((1, hid), lambda i, *_: (0, 0)),
                pl.BlockSpec((1, hid), lambda i, *_: (0, 0)),
                pl.BlockSpec((1, hid), lambda i, *_: (0, 0)),
                pl.BlockSpec((hid, dout), lambda i, *_: (0, 0)),
                pl.BlockSpec((1, dout), lambda i, *_: (0, 0)),
            ],
            out_specs=[pl.BlockSpec((tp, w), lambda i, *_: (i, 0)),
                       pl.BlockSpec((tp, w), lambda i, *_: (i, 0))],
            scratch_shapes=[pltpu.VMEM((tp, 2 * d), jnp.float32)],
        ),
        compiler_params=pltpu.CompilerParams(
            dimension_semantics=("parallel",),
            vmem_limit_bytes=44 * 1024 * 1024),
    )(sub, obj, feats_p, sub[:, None], obj[:, None],
      conf_row, conf_col, w1t, b1, scale, shift, w2t, b2)


def _scatter_combine_kernel(sub_ref, obj_ref, ra_ref, rb_ref, x_ref,
                            conf_row_ref, out_ref, acc_a, acc_b,
                            *, tp, to, np_, dh):
    i = pl.program_id(0)
    w = acc_a.shape[2]

    @pl.when(i == 0)
    def _init():
        acc_a[...] = jnp.zeros(acc_a.shape, jnp.float32)
        acc_b[...] = jnp.zeros(acc_b.shape, jnp.float32)

    @pl.when(i < np_)
    def _scatter_phase():
        base = i * tp
        im = jax.lax.broadcasted_iota(jnp.int32, (8, w), 0)
        for pc in range(tp // 8):
            ca = ra_ref[pc * 8:(pc + 1) * 8, :]
            cb = rb_ref[pc * 8:(pc + 1) * 8, :]
            for k in range(8):
                g = base + pc * 8 + k
                s = sub_ref[g]
                o = obj_ref[g]
                sc = s >> 3
                sl = s & 7
                oc = o >> 3
                ol = o & 7
                add_a = jnp.where(im == sl, pltpu.roll(ca, sl - k, axis=0), 0.0)
                add_b = jnp.where(im == ol, pltpu.roll(cb, ol - k, axis=0), 0.0)
                acc_a[pl.ds(sc, 1)] = acc_a[pl.ds(sc, 1)] + add_a[None]
                acc_b[pl.ds(oc, 1)] = acc_b[pl.ds(oc, 1)] + add_b[None]

    @pl.when(i >= np_)
    def _combine_phase():
        t = i - np_
        cmax = jnp.maximum(jnp.max(conf_row_ref[...]), _SELF_LOGIT)
        w_self = jnp.exp(_SELF_LOGIT - cmax)
        c8 = to // 8
        a_sl = acc_a[pl.ds(t * c8, c8)].reshape(to, w)
        b_sl = acc_b[pl.ds(t * c8, c8)].reshape(to, w)
        num = a_sl[:, :dh] + b_sl[:, :dh]
        den = a_sl[:, dh:dh + 1] + b_sl[:, dh:dh + 1]
        out_ref[...] = (w_self * x_ref[...] + num) / (w_self + den)


def _scatter_combine(sub, obj, ra, rb, feats_p, conf_row, *, tp, to):
    p_pad, w = ra.shape
    o_pad, dh = feats_p.shape
    np_ = p_pad // tp
    no_ = o_pad // to
    return pl.pallas_call(
        functools.partial(_scatter_combine_kernel, tp=tp, to=to, np_=np_,
                          dh=dh),
        out_shape=jax.ShapeDtypeStruct((o_pad, dh), jnp.float32),
        grid_spec=pltpu.PrefetchScalarGridSpec(
            num_scalar_prefetch=2,
            grid=(np_ + no_,),
            in_specs=[
                pl.BlockSpec((tp, w),
                             lambda i, *_: (jnp.minimum(i, np_ - 1), 0)),
                pl.BlockSpec((tp, w),
                             lambda i, *_: (jnp.minimum(i, np_ - 1), 0)),
                pl.BlockSpec((to, dh),
                             lambda i, *_: (jnp.maximum(i - np_, 0), 0)),
                pl.BlockSpec((1, p_pad), lambda i, *_: (0, 0)),
            ],
            out_specs=pl.BlockSpec((to, dh),
                                   lambda i, *_: (jnp.maximum(i - np_, 0), 0)),
            scratch_shapes=[pltpu.VMEM((o_pad // 8, 8, w), jnp.float32),
                            pltpu.VMEM((o_pad // 8, 8, w), jnp.float32)],
        ),
        compiler_params=pltpu.CompilerParams(
            dimension_semantics=("arbitrary",),
            vmem_limit_bytes=52 * 1024 * 1024),
    )(sub, obj, ra, rb, feats_p, conf_row)


def kernel(w1a, w1b, b1, scale, shift, w2t, b2, object_feats, pairs, confidence):
    o, d = object_feats.shape
    p = pairs.shape[0]
    dout = w2t.shape[1]
    dh = dout // 2

    tp = 512 if p >= 512 else _ceil_to(p, 8)
    to = 1024 if o >= 1024 else _ceil_to(o, 8)
    p_pad = _ceil_to(p, tp)
    o_pad = _ceil_to(o, to)

    feats = object_feats.astype(jnp.float32)
    if o_pad != o:
        feats = jnp.concatenate(
            [feats, jnp.zeros((o_pad - o, d), jnp.float32)], axis=0)
    pr = pairs.astype(jnp.int32)
    conf = confidence.astype(jnp.float32)
    if p_pad != p:
        pr = jnp.concatenate(
            [pr, jnp.zeros((p_pad - p, 2), jnp.int32)], axis=0)
        conf = jnp.concatenate(
            [conf, jnp.full((p_pad - p,), _NEG, jnp.float32)], axis=0)
    sub = pr[:, 0]
    obj = pr[:, 1]
    conf_row = conf[None, :]
    conf_col = conf[:, None]
    w1t = jnp.concatenate([w1a, w1b], axis=0)

    ra, rb = _pair_mlp(sub, obj, feats, conf_row, conf_col,
                       w1t, b1, scale, shift, w2t, b2, tp=tp)
    new_p = _scatter_combine(sub, obj, ra, rb, feats, conf_row, tp=tp, to=to)
    new = new_p[:o].astype(object_feats.dtype)
    return new, pairs, confidence
